# Initial kernel scaffold; baseline (speedup 1.0000x reference)
import jax
import jax.numpy as jnp
from jax.experimental import pallas as pl


def kernel(x, edge_index, batch, global_feat, W1_rel, W1_root, b1, W2_rel, W2_root, b2, W3_rel, W3_root, b3, bn1_g, bn1_b, bn2_g, bn2_b, fc1_W, fc1_b, fc2_W, fc2_b, out_W, out_b):
    raise NotImplementedError("write your pallas kernel here")


# speedup vs baseline: 2.2904x; 2.2904x over previous
import functools

import jax
import jax.numpy as jnp
from jax import lax
from jax.experimental import pallas as pl
from jax.experimental.pallas import tpu as pltpu
from jax.experimental.pallas import tpu_sc as plsc

N = 10000
E = 160000
G = 64
EPS = 1e-5

NC = 2
NS = 16
NW = NC * NS
B = 128
EPT = 5120
E_PAD = EPT * NW
STEPS = EPT // B
NROWS = 10240
RPT = NROWS // NS
JUNK = N
ZR = 64

BN = 400
GRID = N // BN


def _segsum_body(x_hbm, src_hbm, dst_hbm, out_hbm,
                 acc, idx_s, idx_d, zbuf, rowbuf, sem):
    cid = lax.axis_index("c")
    sid = lax.axis_index("s")
    tile = cid * NS + sid

    def _zb(i, carry):
        zbuf[i // 8, pl.ds((i % 8) * 16, 16)] = jnp.zeros((16,), jnp.float32)
        return carry
    lax.fori_loop(0, ZR * 8, _zb, 0)
    base = sid * RPT
    for k in range(RPT // ZR):
        pltpu.sync_copy(zbuf, acc.at[pl.ds(base + k * ZR, ZR)])
    plsc.subcore_barrier()

    pltpu.sync_copy(src_hbm.at[pl.ds(tile * STEPS, STEPS)], idx_s)
    pltpu.sync_copy(dst_hbm.at[pl.ds(tile * STEPS, STEPS)], idx_d)

    def _step(j, carry):
        pltpu.async_copy(x_hbm.at[idx_s.at[j]], rowbuf, sem).wait()
        pltpu.sync_copy(rowbuf, acc.at[idx_d.at[j]], add=True)
        return carry
    lax.fori_loop(0, STEPS, _step, 0)

    plsc.subcore_barrier()
    pltpu.sync_copy(acc.at[pl.ds(base, RPT)], out_hbm.at[cid, pl.ds(base, RPT)])


@functools.cache
def _get_segsum128():
    return pl.kernel(
        _segsum_body,
        out_type=jax.ShapeDtypeStruct((NC, NROWS, 128), jnp.float32),
        mesh=plsc.VectorSubcoreMesh(core_axis_name="c", subcore_axis_name="s",
                                    num_cores=NC, num_subcores=NS),
        scratch_types=[
            pltpu.VMEM_SHARED((NROWS, 128), jnp.float32),
            pltpu.VMEM((STEPS, B), jnp.int32),
            pltpu.VMEM((STEPS, B), jnp.int32),
            pltpu.VMEM((ZR, 128), jnp.float32),
            pltpu.VMEM((B, 128), jnp.float32),
            pltpu.SemaphoreType.DMA,
        ],
    )


def _conv_body(p0, p1, x, wr, wt, b, r_ref, s_ref, q_ref):
    i = pl.program_id(0)
    agg = p0[...] + p1[...]
    y = jnp.dot(agg, wr[...], preferred_element_type=jnp.float32)
    y = y + jnp.dot(x[...], wt[...], preferred_element_type=jnp.float32)
    y = y + b[...]
    r = jnp.maximum(y, 0.0)
    r_ref[...] = r

    @pl.when(i == 0)
    def _():
        s_ref[...] = jnp.zeros_like(s_ref)
        q_ref[...] = jnp.zeros_like(q_ref)

    s_ref[...] += jnp.sum(r, axis=0, keepdims=True)
    q_ref[...] += jnp.sum(r * r, axis=0, keepdims=True)


def _conv_call(p0, p1, x, wr, wt, b):
    cin = x.shape[1]
    ca = p0.shape[1]
    h = wr.shape[1]
    return pl.pallas_call(
        _conv_body,
        grid=(GRID,),
        in_specs=[
            pl.BlockSpec((BN, ca), lambda i: (i, 0)),
            pl.BlockSpec((BN, ca), lambda i: (i, 0)),
            pl.BlockSpec((BN, cin), lambda i: (i, 0)),
            pl.BlockSpec((ca, h), lambda i: (0, 0)),
            pl.BlockSpec((cin, h), lambda i: (0, 0)),
            pl.BlockSpec((1, h), lambda i: (0, 0)),
        ],
        out_specs=[
            pl.BlockSpec((BN, h), lambda i: (i, 0)),
            pl.BlockSpec((1, h), lambda i: (0, 0)),
            pl.BlockSpec((1, h), lambda i: (0, 0)),
        ],
        out_shape=[
            jax.ShapeDtypeStruct((N, h), jnp.float32),
            jax.ShapeDtypeStruct((1, h), jnp.float32),
            jax.ShapeDtypeStruct((1, h), jnp.float32),
        ],
    )(p0, p1, x, wr, wt, b)


def _bn_body(r, s, q, gma, bta, h_ref):
    mean = s[...] / N
    var = q[...] / N - mean * mean
    scale = gma[...] * lax.rsqrt(var + EPS)
    h_ref[...] = (r[...] - mean) * scale + bta[...]


def _bn_call(r, s, q, gma, bta):
    h = r.shape[1]
    return pl.pallas_call(
        _bn_body,
        grid=(GRID,),
        in_specs=[
            pl.BlockSpec((BN, h), lambda i: (i, 0)),
            pl.BlockSpec((1, h), lambda i: (0, 0)),
            pl.BlockSpec((1, h), lambda i: (0, 0)),
            pl.BlockSpec((1, h), lambda i: (0, 0)),
            pl.BlockSpec((1, h), lambda i: (0, 0)),
        ],
        out_specs=pl.BlockSpec((BN, h), lambda i: (i, 0)),
        out_shape=jax.ShapeDtypeStruct((N, h), jnp.float32),
    )(r, s, q, gma, bta)


def _pre3_body(h2, w3r, w3t, b3, p3_ref, root_ref):
    p3_ref[...] = jnp.dot(h2[...], w3r[...], preferred_element_type=jnp.float32)
    root_ref[...] = (jnp.dot(h2[...], w3t[...], preferred_element_type=jnp.float32)
                     + b3[...])


def _pre3_call(h2, w3r, w3t, b3):
    h = h2.shape[1]
    ho = w3r.shape[1]
    return pl.pallas_call(
        _pre3_body,
        grid=(GRID,),
        in_specs=[
            pl.BlockSpec((BN, h), lambda i: (i, 0)),
            pl.BlockSpec((h, ho), lambda i: (0, 0)),
            pl.BlockSpec((h, ho), lambda i: (0, 0)),
            pl.BlockSpec((1, ho), lambda i: (0, 0)),
        ],
        out_specs=[
            pl.BlockSpec((BN, ho), lambda i: (i, 0)),
            pl.BlockSpec((BN, ho), lambda i: (i, 0)),
        ],
        out_shape=[
            jax.ShapeDtypeStruct((N, ho), jnp.float32),
            jax.ShapeDtypeStruct((N, ho), jnp.float32),
        ],
    )(h2, w3r, w3t, b3)


def _pool_body(a0, a1, root, bt, ps_ref, cnt_ref):
    i = pl.program_id(0)
    h3 = jnp.maximum(a0[...] + a1[...] + root[...], 0.0)
    onehot = (bt[...] == lax.broadcasted_iota(jnp.int32, (BN, G), 1))
    onehot = onehot.astype(jnp.float32)

    @pl.when(i == 0)
    def _():
        ps_ref[...] = jnp.zeros_like(ps_ref)
        cnt_ref[...] = jnp.zeros_like(cnt_ref)

    ps_ref[...] += lax.dot_general(onehot, h3, (((0,), (0,)), ((), ())),
                                   preferred_element_type=jnp.float32)
    cnt_ref[...] += lax.dot_general(onehot, jnp.ones((BN, 1), jnp.float32),
                                    (((0,), (0,)), ((), ())),
                                    preferred_element_type=jnp.float32)


def _pool_call(a0, a1, root, bt2d):
    ho = root.shape[1]
    return pl.pallas_call(
        _pool_body,
        grid=(GRID,),
        in_specs=[
            pl.BlockSpec((BN, ho), lambda i: (i, 0)),
            pl.BlockSpec((BN, ho), lambda i: (i, 0)),
            pl.BlockSpec((BN, ho), lambda i: (i, 0)),
            pl.BlockSpec((BN, 1), lambda i: (i, 0)),
        ],
        out_specs=[
            pl.BlockSpec((G, ho), lambda i: (0, 0)),
            pl.BlockSpec((G, 1), lambda i: (0, 0)),
        ],
        out_shape=[
            jax.ShapeDtypeStruct((G, ho), jnp.float32),
            jax.ShapeDtypeStruct((G, 1), jnp.float32),
        ],
    )(a0, a1, root, bt2d)


def _mlp_body(ps, cnt, gf, w1a, w1b, fb1, w2, fb2, wo, bo, out_ref):
    pooled = ps[...] / jnp.maximum(cnt[...], 1.0)
    h = jnp.dot(pooled, w1a[...], preferred_element_type=jnp.float32)
    h = h + jnp.dot(gf[...], w1b[...], preferred_element_type=jnp.float32)
    h = jnp.maximum(h + fb1[...], 0.0)
    h = jnp.maximum(jnp.dot(h, w2[...], preferred_element_type=jnp.float32)
                    + fb2[...], 0.0)
    out_ref[...] = (jnp.dot(h, wo[...], preferred_element_type=jnp.float32)
                    + bo[...])


def _mlp_call(ps, cnt, gf, w1a, w1b, fb1, w2, fb2, wo, bo):
    return pl.pallas_call(
        _mlp_body,
        out_shape=jax.ShapeDtypeStruct((G, 1), jnp.float32),
    )(ps, cnt, gf, w1a, w1b, fb1, w2, fb2, wo, bo)


def _segsum(v, src_p, dst_p):
    c = v.shape[1]
    parts0, parts1 = [], []
    for k in range(c // 128):
        o = _get_segsum128()(v[:, k * 128:(k + 1) * 128], src_p, dst_p)
        parts0.append(o[0, :N])
        parts1.append(o[1, :N])
    if len(parts0) == 1:
        return parts0[0], parts1[0]
    return jnp.concatenate(parts0, axis=1), jnp.concatenate(parts1, axis=1)


def kernel(x, edge_index, batch, global_feat,
           W1_rel, W1_root, b1, W2_rel, W2_root, b2, W3_rel, W3_root, b3,
           bn1_g, bn1_b, bn2_g, bn2_b,
           fc1_W, fc1_b, fc2_W, fc2_b, out_W, out_b):
    src = edge_index[0]
    dst = edge_index[1]
    pad = E_PAD - E
    src_p = jnp.concatenate([src, jnp.zeros((pad,), jnp.int32)]).reshape(E_PAD // B, B)
    dst_p = jnp.concatenate([dst, jnp.full((pad,), JUNK, jnp.int32)]).reshape(E_PAD // B, B)

    row = lambda v: v.reshape(1, -1)

    p0, p1 = _segsum(x, src_p, dst_p)
    r1, s1, q1 = _conv_call(p0, p1, x, W1_rel, W1_root, row(b1))
    h1 = _bn_call(r1, s1, q1, row(bn1_g), row(bn1_b))

    p0, p1 = _segsum(h1, src_p, dst_p)
    r2, s2, q2 = _conv_call(p0, p1, h1, W2_rel, W2_root, row(b2))
    h2 = _bn_call(r2, s2, q2, row(bn2_g), row(bn2_b))

    p3, root3 = _pre3_call(h2, W3_rel, W3_root, row(b3))
    a0, a1 = _segsum(p3, src_p, dst_p)
    ps, cnt = _pool_call(a0, a1, root3, batch.reshape(N, 1))

    out = _mlp_call(ps, cnt, global_feat,
                    fc1_W[:256], fc1_W[256:], row(fc1_b),
                    fc2_W, row(fc2_b), out_W, row(out_b))
    return out[:, 0]
